# Initial kernel scaffold; baseline (speedup 1.0000x reference)
import jax
import jax.numpy as jnp
from jax.experimental import pallas as pl


def kernel(x, edge_index, edge_weights, edge_attr, pre_ln_g, pre_ln_b, grp_ln_g, grp_ln_b, Wl, bl, Wr, br, post_ln_g, post_ln_b, et_ln_g, et_ln_b, et_W, et_b, en_g, en_b):
    raise NotImplementedError("write your pallas kernel here")


# speedup vs baseline: 4.4105x; 4.4105x over previous
import functools

import jax
import jax.numpy as jnp
from jax import lax
from jax.experimental import pallas as pl
from jax.experimental.pallas import tpu as pltpu
from jax.experimental.pallas import tpu_sc as plsc

N, E, D, DG, L = 10000, 320000, 128, 64, 4
NPAD = 10240
NC, NS = 2, 16
NW = NC * NS
EPW = E // NW
K = 80
NCHUNK = EPW // K
RPT = NPAD // NS
DW = 16

_mesh = plsc.VectorSubcoreMesh(
    core_axis_name="c", subcore_axis_name="s", num_cores=NC, num_subcores=NS)
_sc_params = pltpu.CompilerParams(use_tc_tiling_on_sc=False)


@functools.partial(
    pl.kernel,
    out_type=jax.ShapeDtypeStruct((NC, NPAD, DG), jnp.float32),
    mesh=_mesh,
    compiler_params=_sc_params,
    scratch_types=[
        pltpu.VMEM((K,), jnp.int32),
        pltpu.VMEM((K,), jnp.int32),
        pltpu.VMEM((K, DG), jnp.float32),
        pltpu.VMEM((RPT, DG), jnp.float32),
        pltpu.VMEM_SHARED((NPAD, DG), jnp.float32),
        pltpu.SemaphoreType.DMA,
    ],
)
def _sc_segsum(src_hbm, dst_hbm, z_hbm, out_hbm, sidx, didx, rows, zbuf, acc, gsem):
    c = lax.axis_index("c")
    s = lax.axis_index("s")
    wid = s * NC + c
    zero16 = jnp.zeros((16,), jnp.float32)

    def _zb(i, carry):
        zbuf[i // (DG // 16), pl.ds((i % (DG // 16)) * 16, 16)] = zero16
        return carry

    lax.fori_loop(0, RPT * (DG // 16), _zb, 0)
    pltpu.sync_copy(zbuf, acc.at[pl.ds(s * RPT, RPT)])
    plsc.subcore_barrier()

    ebase = wid * EPW

    def _chunk(i, carry):
        base = ebase + i * K
        pltpu.sync_copy(src_hbm.at[pl.ds(base, K)], sidx)
        pltpu.sync_copy(dst_hbm.at[pl.ds(base, K)], didx)
        pltpu.async_copy(z_hbm.at[sidx], rows, gsem).wait()
        pltpu.sync_copy(rows, acc.at[didx], add=True)
        return carry

    lax.fori_loop(0, NCHUNK, _chunk, 0)
    plsc.subcore_barrier()
    pltpu.sync_copy(acc.at[pl.ds(s * RPT, RPT)],
                    out_hbm.at[c, pl.ds(s * RPT, RPT)])


@functools.partial(
    pl.kernel,
    out_type=jax.ShapeDtypeStruct((NC, NPAD, DW), jnp.float32),
    mesh=_mesh,
    compiler_params=_sc_params,
    scratch_types=[
        pltpu.VMEM((K,), jnp.int32),
        pltpu.VMEM((K, DW), jnp.float32),
        pltpu.VMEM((RPT, DW), jnp.float32),
        pltpu.VMEM_SHARED((NPAD, DW), jnp.float32),
    ],
)
def _sc_deg(dst_hbm, out_hbm, didx, ones, zbuf, acc):
    c = lax.axis_index("c")
    s = lax.axis_index("s")
    wid = s * NC + c
    zero16 = jnp.zeros((16,), jnp.float32)
    one16 = jnp.ones((16,), jnp.float32)

    def _fill(i, carry):
        ones[i, :] = one16
        return carry

    lax.fori_loop(0, K, _fill, 0)

    def _zb(i, carry):
        zbuf[i, :] = zero16
        return carry

    lax.fori_loop(0, RPT, _zb, 0)
    pltpu.sync_copy(zbuf, acc.at[pl.ds(s * RPT, RPT)])
    plsc.subcore_barrier()

    ebase = wid * EPW

    def _chunk(i, carry):
        base = ebase + i * K
        pltpu.sync_copy(dst_hbm.at[pl.ds(base, K)], didx)
        pltpu.sync_copy(ones, acc.at[didx], add=True)
        return carry

    lax.fori_loop(0, NCHUNK, _chunk, 0)
    plsc.subcore_barrier()
    pltpu.sync_copy(acc.at[pl.ds(s * RPT, RPT)],
                    out_hbm.at[c, pl.ds(s * RPT, RPT)])


def _ln(x, g, b, eps=1e-5):
    m = jnp.mean(x, axis=-1, keepdims=True)
    v = jnp.mean((x - m) ** 2, axis=-1, keepdims=True)
    return (x - m) / jnp.sqrt(v + eps) * g + b


_SQRT_HALF = 0.7071067811865476


def _gelu(x):
    return 0.5 * x * (1.0 + lax.erf(x * _SQRT_HALF))


def _pre_body(x_ref, pg, pb, gg, gb, h_ref, z0_ref):
    x = x_ref[...]
    h = _gelu(_ln(x, pg[...], pb[...]))
    h_ref[...] = h
    z0_ref[...] = jax.nn.relu(_ln(h[:, DG:], gg[...], gb[...]))


_tc_pre = pl.pallas_call(
    _pre_body,
    out_shape=(jax.ShapeDtypeStruct((N, D), jnp.float32),
               jax.ShapeDtypeStruct((N, DG), jnp.float32)),
)


def _agg_from_partials(aggp_ref, degp_ref):
    agg = aggp_ref[0, :N, :] + aggp_ref[1, :N, :]
    deg = degp_ref[0, :N, :] + degp_ref[1, :N, :]
    deg = jnp.maximum(deg, 1.0)
    return agg / deg[:, :1]


def _mid_body(h_ref, z_ref, aggp_ref, degp_ref, wl, bl, wr, br, gg, gb,
              y0_ref, z1_ref):
    z = z_ref[...]
    agg = _agg_from_partials(aggp_ref, degp_ref)
    conv = (jnp.dot(agg, wl[...], preferred_element_type=jnp.float32) + bl[...]
            + jnp.dot(z, wr[...], preferred_element_type=jnp.float32) + br[...])
    y0 = h_ref[:, :DG] + conv
    y0_ref[...] = y0
    z1_ref[...] = jax.nn.relu(_ln(y0, gg[...], gb[...]))


_tc_mid = pl.pallas_call(
    _mid_body,
    out_shape=(jax.ShapeDtypeStruct((N, DG), jnp.float32),
               jax.ShapeDtypeStruct((N, DG), jnp.float32)),
)


def _post_body(h_ref, y0_ref, z1_ref, aggp_ref, degp_ref, wl, bl, wr, br,
               xres_ref, pg, pb, out_ref, *, final):
    z1 = z1_ref[...]
    agg = _agg_from_partials(aggp_ref, degp_ref)
    conv = (jnp.dot(agg, wl[...], preferred_element_type=jnp.float32) + bl[...]
            + jnp.dot(z1, wr[...], preferred_element_type=jnp.float32) + br[...])
    y1 = h_ref[:, DG:] + conv
    hcat = jnp.concatenate([y0_ref[...], y1], axis=-1)
    xn = _ln(hcat + xres_ref[...], pg[...], pb[...])
    out_ref[...] = _gelu(xn) if final else xn


_tc_post = pl.pallas_call(
    functools.partial(_post_body, final=False),
    out_shape=jax.ShapeDtypeStruct((N, D), jnp.float32),
)
_tc_post_final = pl.pallas_call(
    functools.partial(_post_body, final=True),
    out_shape=jax.ShapeDtypeStruct((N, D), jnp.float32),
)


def kernel(x, edge_index, edge_weights, edge_attr, pre_ln_g, pre_ln_b,
           grp_ln_g, grp_ln_b, Wl, bl, Wr, br, post_ln_g, post_ln_b,
           et_ln_g, et_ln_b, et_W, et_b, en_g, en_b):
    src = edge_index[0]
    dst = edge_index[1]
    degp = _sc_deg(dst)
    Wlt = jnp.swapaxes(Wl, -1, -2)
    Wrt = jnp.swapaxes(Wr, -1, -2)
    for l in range(L):
        xres = x
        h, z0 = _tc_pre(x, pre_ln_g[l][None], pre_ln_b[l][None],
                        grp_ln_g[l, 0][None], grp_ln_b[l, 0][None])
        aggp0 = _sc_segsum(src, dst, z0)
        y0, z1 = _tc_mid(h, z0, aggp0, degp, Wlt[l, 0], bl[l, 0][None],
                         Wrt[l, 0], br[l, 0][None],
                         grp_ln_g[l, 1][None], grp_ln_b[l, 1][None])
        aggp1 = _sc_segsum(src, dst, z1)
        post = _tc_post_final if l == L - 1 else _tc_post
        x = post(h, y0, z1, aggp1, degp, Wlt[l, 1], bl[l, 1][None],
                 Wrt[l, 1], br[l, 1][None], xres,
                 post_ln_g[l][None], post_ln_b[l][None])
    return x


# speedup vs baseline: 11.8412x; 2.6848x over previous
import functools

import jax
import jax.numpy as jnp
from jax import lax
from jax.experimental import pallas as pl
from jax.experimental.pallas import tpu as pltpu
from jax.experimental.pallas import tpu_sc as plsc

N, E, D, DG, L = 10000, 320000, 128, 64, 4
NPAD = 10240
NC, NS = 2, 16
NW = NC * NS
EPW = E // NW
K = 125
NCHUNK = EPW // K
NBUF = 5
NROUND = NCHUNK // NBUF
RPT = NPAD // NS
DW = 16

_mesh = plsc.VectorSubcoreMesh(
    core_axis_name="c", subcore_axis_name="s", num_cores=NC, num_subcores=NS)
_sc_params = pltpu.CompilerParams(use_tc_tiling_on_sc=False)


@functools.partial(
    pl.kernel,
    out_type=jax.ShapeDtypeStruct((NC, NPAD, DG), jnp.float32),
    mesh=_mesh,
    compiler_params=_sc_params,
    scratch_types=[
        pltpu.VMEM((NCHUNK, K), jnp.int32),
        pltpu.VMEM((NCHUNK, K), jnp.int32),
        pltpu.VMEM((NBUF, K, DG), jnp.float32),
        pltpu.VMEM((64, DG), jnp.float32),
        pltpu.VMEM_SHARED((NPAD, DG), jnp.float32),
        pltpu.SemaphoreType.DMA((NBUF,)),
        pltpu.SemaphoreType.DMA((NBUF,)),
        pltpu.SemaphoreType.DMA,
        pltpu.SemaphoreType.DMA,
    ],
)
def _sc_segsum(src_hbm, dst_hbm, z_hbm, out_hbm, sidx, didx, rows, zbuf, acc,
               gsem, ssem, isem, zsem):
    c = lax.axis_index("c")
    s = lax.axis_index("s")
    wid = s * NC + c
    zero16 = jnp.zeros((16,), jnp.float32)

    ild0 = pltpu.async_copy(src_hbm.at[pl.ds(wid * NCHUNK, NCHUNK)], sidx, isem)
    ild1 = pltpu.async_copy(dst_hbm.at[pl.ds(wid * NCHUNK, NCHUNK)], didx, isem)

    def _zb(i, carry):
        zbuf[i // (DG // 16), pl.ds((i % (DG // 16)) * 16, 16)] = zero16
        return carry

    lax.fori_loop(0, 64 * (DG // 16), _zb, 0)

    def _zc(j, carry):
        pltpu.async_copy(zbuf, acc.at[pl.ds(s * RPT + j * 64, 64)], zsem)
        return carry

    lax.fori_loop(0, RPT // 64, _zc, 0)

    def _zd(j, carry):
        pltpu.make_async_copy(zbuf, acc.at[pl.ds(0, 64)], zsem).wait()
        return carry

    lax.fori_loop(0, RPT // 64, _zd, 0)
    ild0.wait()
    ild1.wait()
    plsc.subcore_barrier()

    for b in range(NBUF):
        pltpu.async_copy(z_hbm.at[sidx.at[b]], rows.at[b], gsem.at[b])

    def _round(g, carry):
        descs = []
        for b in range(NBUF):
            pltpu.make_async_copy(z_hbm.at[pl.ds(0, K)], rows.at[b],
                                  gsem.at[b]).wait()
            descs.append(pltpu.async_copy(rows.at[b], acc.at[didx.at[g * NBUF + b]],
                                          ssem.at[b], add=True))
        for b in range(NBUF):
            descs[b].wait()

            @pl.when(g + 1 < NROUND)
            def _():
                pltpu.async_copy(z_hbm.at[sidx.at[(g + 1) * NBUF + b]],
                                 rows.at[b], gsem.at[b])
        return carry

    lax.fori_loop(0, NROUND, _round, 0)
    plsc.subcore_barrier()
    pltpu.sync_copy(acc.at[pl.ds(s * RPT, RPT)],
                    out_hbm.at[c, pl.ds(s * RPT, RPT)])


@functools.partial(
    pl.kernel,
    out_type=jax.ShapeDtypeStruct((NC, NPAD, DW), jnp.float32),
    mesh=_mesh,
    compiler_params=_sc_params,
    scratch_types=[
        pltpu.VMEM((NCHUNK, K), jnp.int32),
        pltpu.VMEM((K, DW), jnp.float32),
        pltpu.VMEM((RPT, DW), jnp.float32),
        pltpu.VMEM_SHARED((NPAD, DW), jnp.float32),
        pltpu.SemaphoreType.DMA,
        pltpu.SemaphoreType.DMA,
    ],
)
def _sc_deg(dst_hbm, out_hbm, didx, ones, zbuf, acc, ssem, isem):
    c = lax.axis_index("c")
    s = lax.axis_index("s")
    wid = s * NC + c
    zero16 = jnp.zeros((16,), jnp.float32)
    one16 = jnp.ones((16,), jnp.float32)

    ild = pltpu.async_copy(dst_hbm.at[pl.ds(wid * NCHUNK, NCHUNK)], didx, isem)

    def _fill(i, carry):
        ones[i, :] = one16
        return carry

    lax.fori_loop(0, K, _fill, 0)

    def _zb(i, carry):
        zbuf[i, :] = zero16
        return carry

    lax.fori_loop(0, RPT, _zb, 0)
    pltpu.sync_copy(zbuf, acc.at[pl.ds(s * RPT, RPT)])
    ild.wait()
    plsc.subcore_barrier()

    def _fire(i, carry):
        pltpu.async_copy(ones, acc.at[didx.at[i]], ssem, add=True)
        return carry

    lax.fori_loop(0, NCHUNK, _fire, 0)

    def _drain(i, carry):
        pltpu.make_async_copy(ones, acc.at[didx.at[0]], ssem).wait()
        return carry

    lax.fori_loop(0, NCHUNK, _drain, 0)
    plsc.subcore_barrier()
    pltpu.sync_copy(acc.at[pl.ds(s * RPT, RPT)],
                    out_hbm.at[c, pl.ds(s * RPT, RPT)])


def _ln(x, g, b, eps=1e-5):
    m = jnp.mean(x, axis=-1, keepdims=True)
    v = jnp.mean((x - m) ** 2, axis=-1, keepdims=True)
    return (x - m) / jnp.sqrt(v + eps) * g + b


_SQRT_HALF = 0.7071067811865476


def _gelu(x):
    return 0.5 * x * (1.0 + lax.erf(x * _SQRT_HALF))


def _pre_body(x_ref, pg, pb, gg, gb, h_ref, z0_ref):
    x = x_ref[...]
    h = _gelu(_ln(x, pg[...], pb[...]))
    h_ref[...] = h
    z0_ref[...] = jax.nn.relu(_ln(h[:, DG:], gg[...], gb[...]))


_tc_pre = pl.pallas_call(
    _pre_body,
    out_shape=(jax.ShapeDtypeStruct((N, D), jnp.float32),
               jax.ShapeDtypeStruct((N, DG), jnp.float32)),
)


def _agg_from_partials(aggp_ref, degp_ref):
    agg = aggp_ref[0, :N, :] + aggp_ref[1, :N, :]
    deg = degp_ref[0, :N, :] + degp_ref[1, :N, :]
    deg = jnp.maximum(deg, 1.0)
    return agg / deg[:, :1]


def _mid_body(h_ref, z_ref, aggp_ref, degp_ref, wl, bl, wr, br, gg, gb,
              y0_ref, z1_ref):
    z = z_ref[...]
    agg = _agg_from_partials(aggp_ref, degp_ref)
    conv = (jnp.dot(agg, wl[...], preferred_element_type=jnp.float32) + bl[...]
            + jnp.dot(z, wr[...], preferred_element_type=jnp.float32) + br[...])
    y0 = h_ref[:, :DG] + conv
    y0_ref[...] = y0
    z1_ref[...] = jax.nn.relu(_ln(y0, gg[...], gb[...]))


_tc_mid = pl.pallas_call(
    _mid_body,
    out_shape=(jax.ShapeDtypeStruct((N, DG), jnp.float32),
               jax.ShapeDtypeStruct((N, DG), jnp.float32)),
)


def _post_body(h_ref, y0_ref, z1_ref, aggp_ref, degp_ref, wl, bl, wr, br,
               xres_ref, pg, pb, out_ref):
    z1 = z1_ref[...]
    agg = _agg_from_partials(aggp_ref, degp_ref)
    conv = (jnp.dot(agg, wl[...], preferred_element_type=jnp.float32) + bl[...]
            + jnp.dot(z1, wr[...], preferred_element_type=jnp.float32) + br[...])
    y1 = h_ref[:, DG:] + conv
    hcat = jnp.concatenate([y0_ref[...], y1], axis=-1)
    out_ref[...] = _ln(hcat + xres_ref[...], pg[...], pb[...])


_tc_post = pl.pallas_call(
    _post_body,
    out_shape=jax.ShapeDtypeStruct((N, D), jnp.float32),
)


def _gelu_body(x_ref, out_ref):
    out_ref[...] = _gelu(x_ref[...])


_tc_gelu = pl.pallas_call(
    _gelu_body,
    out_shape=jax.ShapeDtypeStruct((N, D), jnp.float32),
)


def kernel(x, edge_index, edge_weights, edge_attr, pre_ln_g, pre_ln_b,
           grp_ln_g, grp_ln_b, Wl, bl, Wr, br, post_ln_g, post_ln_b,
           et_ln_g, et_ln_b, et_W, et_b, en_g, en_b):
    src = edge_index[0].reshape(E // K, K)
    dst = edge_index[1].reshape(E // K, K)
    degp = _sc_deg(dst)
    Wlt = jnp.swapaxes(Wl, -1, -2)
    Wrt = jnp.swapaxes(Wr, -1, -2)

    def _interleave(a0, a1):
        return jnp.stack([a0, a1], axis=1).reshape((2 * L,) + a0.shape[1:])

    roll = lambda a: jnp.roll(a, -1, axis=0)
    wlk = _interleave(Wlt[:, 0], Wlt[:, 1])
    blk = _interleave(bl[:, 0], bl[:, 1])[:, None]
    wrk = _interleave(Wrt[:, 0], Wrt[:, 1])
    brk = _interleave(br[:, 0], br[:, 1])[:, None]
    gnk_g = _interleave(grp_ln_g[:, 1], roll(grp_ln_g[:, 0]))[:, None]
    gnk_b = _interleave(grp_ln_b[:, 1], roll(grp_ln_b[:, 0]))[:, None]
    zeroD = jnp.zeros((L, D), jnp.float32)
    post_gk = _interleave(zeroD, post_ln_g)[:, None]
    post_bk = _interleave(zeroD, post_ln_b)[:, None]
    pre_gk = _interleave(zeroD, roll(pre_ln_g))[:, None]
    pre_bk = _interleave(zeroD, roll(pre_ln_b))[:, None]
    is_mid = (jnp.arange(2 * L) % 2) == 0

    h0, z0 = _tc_pre(x, pre_ln_g[0][None], pre_ln_b[0][None],
                     grp_ln_g[0, 0][None], grp_ln_b[0, 0][None])
    carry0 = (x, h0, jnp.zeros((N, DG), jnp.float32), z0)

    def group_step(carry, w):
        xres, h, y0, z = carry
        (mid, wlc, blc, wrc, brc, gng, gnb, og, ob, pg, pb) = w
        aggp = _sc_segsum(src, dst, z)

        def mid_branch(ops):
            xres, h, y0, z, aggp = ops
            y0n, zn = _tc_mid(h, z, aggp, degp, wlc, blc, wrc, brc, gng, gnb)
            return (xres, h, y0n, zn)

        def postpre_branch(ops):
            xres, h, y0, z, aggp = ops
            xn = _tc_post(h, y0, z, aggp, degp, wlc, blc, wrc, brc,
                          xres, og, ob)
            hn, zn = _tc_pre(xn, pg, pb, gng, gnb)
            return (xn, hn, y0, zn)

        carry = lax.cond(mid, mid_branch, postpre_branch,
                         (xres, h, y0, z, aggp))
        return carry, None

    (x, _, _, _), _ = lax.scan(
        group_step, carry0,
        (is_mid, wlk, blk, wrk, brk, gnk_g, gnk_b,
         post_gk, post_bk, pre_gk, pre_bk))
    return _tc_gelu(x)


# speedup vs baseline: 13.0907x; 1.1055x over previous
import functools

import jax
import jax.numpy as jnp
from jax import lax
from jax.experimental import pallas as pl
from jax.experimental.pallas import tpu as pltpu
from jax.experimental.pallas import tpu_sc as plsc

N, E, D, DG, L = 10000, 320000, 128, 64, 4
NPAD = 10240
NC, NS = 2, 16
NW = NC * NS
EPW = E // NW
K = 125
NCHUNK = EPW // K
NBUF = 5
NROUND = NCHUNK // NBUF
RPT = NPAD // NS
DW = 16

_mesh = plsc.VectorSubcoreMesh(
    core_axis_name="c", subcore_axis_name="s", num_cores=NC, num_subcores=NS)
_sc_params = pltpu.CompilerParams(use_tc_tiling_on_sc=False)


@functools.partial(
    pl.kernel,
    out_type=jax.ShapeDtypeStruct((NC, NPAD, DG), jnp.float32),
    mesh=_mesh,
    compiler_params=_sc_params,
    scratch_types=[
        pltpu.VMEM((NCHUNK, K), jnp.int32),
        pltpu.VMEM((NCHUNK, K), jnp.int32),
        pltpu.VMEM((NBUF, K, DG), jnp.float32),
        pltpu.VMEM((64, DG), jnp.float32),
        pltpu.VMEM_SHARED((NPAD, DG), jnp.float32),
        pltpu.SemaphoreType.DMA((NBUF,)),
        pltpu.SemaphoreType.DMA((NBUF,)),
        pltpu.SemaphoreType.DMA,
        pltpu.SemaphoreType.DMA,
    ],
)
def _sc_segsum(src_hbm, dst_hbm, z_hbm, out_hbm, sidx, didx, rows, zbuf, acc,
               gsem, ssem, isem, zsem):
    c = lax.axis_index("c")
    s = lax.axis_index("s")
    wid = s * NC + c
    zero16 = jnp.zeros((16,), jnp.float32)

    ild0 = pltpu.async_copy(src_hbm.at[pl.ds(wid * NCHUNK, NCHUNK)], sidx, isem)
    ild1 = pltpu.async_copy(dst_hbm.at[pl.ds(wid * NCHUNK, NCHUNK)], didx, isem)

    def _zb(i, carry):
        zbuf[i // (DG // 16), pl.ds((i % (DG // 16)) * 16, 16)] = zero16
        return carry

    lax.fori_loop(0, 64 * (DG // 16), _zb, 0)

    def _zc(j, carry):
        pltpu.async_copy(zbuf, acc.at[pl.ds(s * RPT + j * 64, 64)], zsem)
        return carry

    lax.fori_loop(0, RPT // 64, _zc, 0)

    def _zd(j, carry):
        pltpu.make_async_copy(zbuf, acc.at[pl.ds(0, 64)], zsem).wait()
        return carry

    lax.fori_loop(0, RPT // 64, _zd, 0)
    ild0.wait()
    ild1.wait()
    plsc.subcore_barrier()

    for b in range(NBUF):
        pltpu.async_copy(z_hbm.at[sidx.at[b]], rows.at[b], gsem.at[b])

    def _round(g, carry):
        descs = []
        for b in range(NBUF):
            pltpu.make_async_copy(z_hbm.at[pl.ds(0, K)], rows.at[b],
                                  gsem.at[b]).wait()
            descs.append(pltpu.async_copy(rows.at[b], acc.at[didx.at[g * NBUF + b]],
                                          ssem.at[b], add=True))
        for b in range(NBUF):
            descs[b].wait()

            @pl.when(g + 1 < NROUND)
            def _():
                pltpu.async_copy(z_hbm.at[sidx.at[(g + 1) * NBUF + b]],
                                 rows.at[b], gsem.at[b])
        return carry

    lax.fori_loop(0, NROUND, _round, 0)
    plsc.subcore_barrier()
    pltpu.sync_copy(acc.at[pl.ds(s * RPT, RPT)],
                    out_hbm.at[c, pl.ds(s * RPT, RPT)])


@functools.partial(
    pl.kernel,
    out_type=jax.ShapeDtypeStruct((NC, NPAD, DW), jnp.float32),
    mesh=_mesh,
    compiler_params=_sc_params,
    scratch_types=[
        pltpu.VMEM((NCHUNK, K), jnp.int32),
        pltpu.VMEM((K, DW), jnp.float32),
        pltpu.VMEM((RPT, DW), jnp.float32),
        pltpu.VMEM_SHARED((NPAD, DW), jnp.float32),
        pltpu.SemaphoreType.DMA,
        pltpu.SemaphoreType.DMA,
    ],
)
def _sc_deg(dst_hbm, out_hbm, didx, ones, zbuf, acc, ssem, isem):
    c = lax.axis_index("c")
    s = lax.axis_index("s")
    wid = s * NC + c
    zero16 = jnp.zeros((16,), jnp.float32)
    one16 = jnp.ones((16,), jnp.float32)

    ild = pltpu.async_copy(dst_hbm.at[pl.ds(wid * NCHUNK, NCHUNK)], didx, isem)

    def _fill(i, carry):
        ones[i, :] = one16
        return carry

    lax.fori_loop(0, K, _fill, 0)

    def _zb(i, carry):
        zbuf[i, :] = zero16
        return carry

    lax.fori_loop(0, RPT, _zb, 0)
    pltpu.sync_copy(zbuf, acc.at[pl.ds(s * RPT, RPT)])
    ild.wait()
    plsc.subcore_barrier()

    def _fire(i, carry):
        pltpu.async_copy(ones, acc.at[didx.at[i]], ssem, add=True)
        return carry

    lax.fori_loop(0, NCHUNK, _fire, 0)

    def _drain(i, carry):
        pltpu.make_async_copy(ones, acc.at[didx.at[0]], ssem).wait()
        return carry

    lax.fori_loop(0, NCHUNK, _drain, 0)
    plsc.subcore_barrier()
    pltpu.sync_copy(acc.at[pl.ds(s * RPT, RPT)],
                    out_hbm.at[c, pl.ds(s * RPT, RPT)])


def _ln(x, g, b, eps=1e-5):
    m = jnp.mean(x, axis=-1, keepdims=True)
    v = jnp.mean((x - m) ** 2, axis=-1, keepdims=True)
    return (x - m) / jnp.sqrt(v + eps) * g + b


_SQRT_HALF = 0.7071067811865476


def _gelu(x):
    return 0.5 * x * (1.0 + lax.erf(x * _SQRT_HALF))


def _pre_body(x_ref, pg, pb, gg, gb, h_ref, z0_ref):
    x = x_ref[...]
    h = _gelu(_ln(x, pg[...], pb[...]))
    h_ref[...] = h
    z0_ref[...] = jax.nn.relu(_ln(h[:, DG:], gg[...], gb[...]))


_tc_pre = pl.pallas_call(
    _pre_body,
    out_shape=(jax.ShapeDtypeStruct((N, D), jnp.float32),
               jax.ShapeDtypeStruct((N, DG), jnp.float32)),
)


def _agg_from_partials(aggp_ref, degp_ref):
    agg = aggp_ref[0, :N, :] + aggp_ref[1, :N, :]
    deg = degp_ref[0, :N, :] + degp_ref[1, :N, :]
    deg = jnp.maximum(deg, 1.0)
    return agg / deg[:, :1]


def _mid_body(h_ref, z_ref, aggp_ref, degp_ref, wl, bl, wr, br, gg, gb,
              y0_ref, z1_ref):
    z = z_ref[...]
    agg = _agg_from_partials(aggp_ref, degp_ref)
    conv = (jnp.dot(agg, wl[...], preferred_element_type=jnp.float32) + bl[...]
            + jnp.dot(z, wr[...], preferred_element_type=jnp.float32) + br[...])
    y0 = h_ref[:, :DG] + conv
    y0_ref[...] = y0
    z1_ref[...] = jax.nn.relu(_ln(y0, gg[...], gb[...]))


_tc_mid = pl.pallas_call(
    _mid_body,
    out_shape=(jax.ShapeDtypeStruct((N, DG), jnp.float32),
               jax.ShapeDtypeStruct((N, DG), jnp.float32)),
)


def _post_body(h_ref, y0_ref, z1_ref, aggp_ref, degp_ref, wl, bl, wr, br,
               xres_ref, pg, pb, out_ref):
    z1 = z1_ref[...]
    agg = _agg_from_partials(aggp_ref, degp_ref)
    conv = (jnp.dot(agg, wl[...], preferred_element_type=jnp.float32) + bl[...]
            + jnp.dot(z1, wr[...], preferred_element_type=jnp.float32) + br[...])
    y1 = h_ref[:, DG:] + conv
    hcat = jnp.concatenate([y0_ref[...], y1], axis=-1)
    out_ref[...] = _ln(hcat + xres_ref[...], pg[...], pb[...])


_tc_post = pl.pallas_call(
    _post_body,
    out_shape=jax.ShapeDtypeStruct((N, D), jnp.float32),
)


def _gelu_body(x_ref, out_ref):
    out_ref[...] = _gelu(x_ref[...])


_tc_gelu = pl.pallas_call(
    _gelu_body,
    out_shape=jax.ShapeDtypeStruct((N, D), jnp.float32),
)


def kernel(x, edge_index, edge_weights, edge_attr, pre_ln_g, pre_ln_b,
           grp_ln_g, grp_ln_b, Wl, bl, Wr, br, post_ln_g, post_ln_b,
           et_ln_g, et_ln_b, et_W, et_b, en_g, en_b):
    src = edge_index[0].reshape(E // K, K)
    dst = edge_index[1].reshape(E // K, K)
    degp = _sc_deg(dst)
    Wlt = jnp.swapaxes(Wl, -1, -2)
    Wrt = jnp.swapaxes(Wr, -1, -2)
    for l in range(L):
        xres = x
        h, z0 = _tc_pre(x, pre_ln_g[l][None], pre_ln_b[l][None],
                        grp_ln_g[l, 0][None], grp_ln_b[l, 0][None])
        aggp0 = _sc_segsum(src, dst, z0)
        y0, z1 = _tc_mid(h, z0, aggp0, degp, Wlt[l, 0], bl[l, 0][None],
                         Wrt[l, 0], br[l, 0][None],
                         grp_ln_g[l, 1][None], grp_ln_b[l, 1][None])
        aggp1 = _sc_segsum(src, dst, z1)
        x = _tc_post(h, y0, z1, aggp1, degp, Wlt[l, 1], bl[l, 1][None],
                     Wrt[l, 1], br[l, 1][None], xres,
                     post_ln_g[l][None], post_ln_b[l][None])
    return _tc_gelu(x)


# speedup vs baseline: 14.0356x; 1.0722x over previous
import functools

import jax
import jax.numpy as jnp
from jax import lax
from jax.experimental import pallas as pl
from jax.experimental.pallas import tpu as pltpu
from jax.experimental.pallas import tpu_sc as plsc

N, E, D, DG, L = 10000, 320000, 128, 64, 4
NPAD = 10240
NC, NS = 2, 16
NW = NC * NS
EPW = E // NW
K = 125
NCHUNK = EPW // K
NBUF = 5
NROUND = NCHUNK // NBUF
RPT = NPAD // NS
DW = 16

_mesh = plsc.VectorSubcoreMesh(
    core_axis_name="c", subcore_axis_name="s", num_cores=NC, num_subcores=NS)
_sc_params = pltpu.CompilerParams(use_tc_tiling_on_sc=False)


@functools.partial(
    pl.kernel,
    out_type=jax.ShapeDtypeStruct((NC, NPAD, DG), jnp.float32),
    mesh=_mesh,
    compiler_params=_sc_params,
    scratch_types=[
        pltpu.VMEM((NCHUNK, K), jnp.int32),
        pltpu.VMEM((NCHUNK, K), jnp.int32),
        pltpu.VMEM((NBUF, K, DG), jnp.float32),
        pltpu.VMEM((64, DG), jnp.float32),
        pltpu.VMEM_SHARED((NPAD, DG), jnp.float32),
        pltpu.SemaphoreType.DMA((NBUF,)),
        pltpu.SemaphoreType.DMA((NBUF,)),
        pltpu.SemaphoreType.DMA,
        pltpu.SemaphoreType.DMA,
    ],
)
def _sc_segsum(src_hbm, dst_hbm, z_hbm, out_hbm, sidx, didx, rows, zbuf, acc,
               gsem, ssem, isem, zsem):
    c = lax.axis_index("c")
    s = lax.axis_index("s")
    wid = s * NC + c
    zero16 = jnp.zeros((16,), jnp.float32)

    ild0 = pltpu.async_copy(src_hbm.at[pl.ds(wid * NCHUNK, NCHUNK)], sidx, isem)
    ild1 = pltpu.async_copy(dst_hbm.at[pl.ds(wid * NCHUNK, NCHUNK)], didx, isem)

    def _zb(i, carry):
        zbuf[i // (DG // 16), pl.ds((i % (DG // 16)) * 16, 16)] = zero16
        return carry

    lax.fori_loop(0, 64 * (DG // 16), _zb, 0)

    def _zc(j, carry):
        pltpu.async_copy(zbuf, acc.at[pl.ds(s * RPT + j * 64, 64)], zsem)
        return carry

    lax.fori_loop(0, RPT // 64, _zc, 0)

    def _zd(j, carry):
        pltpu.make_async_copy(zbuf, acc.at[pl.ds(0, 64)], zsem).wait()
        return carry

    lax.fori_loop(0, RPT // 64, _zd, 0)
    ild0.wait()
    ild1.wait()
    plsc.subcore_barrier()

    for b in range(NBUF):
        pltpu.async_copy(z_hbm.at[sidx.at[b]], rows.at[b], gsem.at[b])

    def _round(g, carry):
        descs = []
        for b in range(NBUF):
            pltpu.make_async_copy(z_hbm.at[pl.ds(0, K)], rows.at[b],
                                  gsem.at[b]).wait()
            descs.append(pltpu.async_copy(rows.at[b], acc.at[didx.at[g * NBUF + b]],
                                          ssem.at[b], add=True))
        for b in range(NBUF):
            descs[b].wait()

            @pl.when(g + 1 < NROUND)
            def _():
                pltpu.async_copy(z_hbm.at[sidx.at[(g + 1) * NBUF + b]],
                                 rows.at[b], gsem.at[b])
        return carry

    lax.fori_loop(0, NROUND, _round, 0)
    plsc.subcore_barrier()
    pltpu.sync_copy(acc.at[pl.ds(s * RPT, RPT)],
                    out_hbm.at[c, pl.ds(s * RPT, RPT)])


@functools.partial(
    pl.kernel,
    out_type=(jax.ShapeDtypeStruct((NC, NPAD, DG), jnp.float32),
              jax.ShapeDtypeStruct((NC, NPAD, DW), jnp.float32)),
    mesh=_mesh,
    compiler_params=_sc_params,
    scratch_types=[
        pltpu.VMEM((NCHUNK, K), jnp.int32),
        pltpu.VMEM((NCHUNK, K), jnp.int32),
        pltpu.VMEM((NBUF, K, DG), jnp.float32),
        pltpu.VMEM((64, DG), jnp.float32),
        pltpu.VMEM((K, DW), jnp.float32),
        pltpu.VMEM((64, DW), jnp.float32),
        pltpu.VMEM_SHARED((NPAD, DG), jnp.float32),
        pltpu.VMEM_SHARED((NPAD, DW), jnp.float32),
        pltpu.SemaphoreType.DMA((NBUF,)),
        pltpu.SemaphoreType.DMA((NBUF,)),
        pltpu.SemaphoreType.DMA,
        pltpu.SemaphoreType.DMA,
        pltpu.SemaphoreType.DMA,
    ],
)
def _sc_segsum_deg(src_hbm, dst_hbm, z_hbm, out_hbm, deg_hbm, sidx, didx, rows,
                   zbuf, ones, zbuf2, acc, dacc, gsem, ssem, isem, zsem, dsem):
    c = lax.axis_index("c")
    s = lax.axis_index("s")
    wid = s * NC + c
    zero16 = jnp.zeros((16,), jnp.float32)
    one16 = jnp.ones((16,), jnp.float32)

    ild0 = pltpu.async_copy(src_hbm.at[pl.ds(wid * NCHUNK, NCHUNK)], sidx, isem)
    ild1 = pltpu.async_copy(dst_hbm.at[pl.ds(wid * NCHUNK, NCHUNK)], didx, isem)

    def _zb(i, carry):
        zbuf[i // (DG // 16), pl.ds((i % (DG // 16)) * 16, 16)] = zero16
        return carry

    lax.fori_loop(0, 64 * (DG // 16), _zb, 0)

    def _fill(i, carry):
        ones[i, :] = one16
        zbuf2[i % 64, :] = zero16
        return carry

    lax.fori_loop(0, K, _fill, 0)

    def _zc(j, carry):
        pltpu.async_copy(zbuf, acc.at[pl.ds(s * RPT + j * 64, 64)], zsem)
        pltpu.async_copy(zbuf2, dacc.at[pl.ds(s * RPT + j * 64, 64)], zsem)
        return carry

    lax.fori_loop(0, RPT // 64, _zc, 0)

    def _zd(j, carry):
        pltpu.make_async_copy(zbuf, acc.at[pl.ds(0, 64)], zsem).wait()
        pltpu.make_async_copy(zbuf2, dacc.at[pl.ds(0, 64)], zsem).wait()
        return carry

    lax.fori_loop(0, RPT // 64, _zd, 0)
    ild0.wait()
    ild1.wait()
    plsc.subcore_barrier()

    for b in range(NBUF):
        pltpu.async_copy(z_hbm.at[sidx.at[b]], rows.at[b], gsem.at[b])

    def _round(g, carry):
        descs = []
        for b in range(NBUF):
            i = g * NBUF + b
            pltpu.make_async_copy(z_hbm.at[pl.ds(0, K)], rows.at[b],
                                  gsem.at[b]).wait()
            descs.append(pltpu.async_copy(rows.at[b], acc.at[didx.at[i]],
                                          ssem.at[b], add=True))
            pltpu.async_copy(ones, dacc.at[didx.at[i]], dsem, add=True)
        for b in range(NBUF):
            descs[b].wait()

            @pl.when(g + 1 < NROUND)
            def _():
                pltpu.async_copy(z_hbm.at[sidx.at[(g + 1) * NBUF + b]],
                                 rows.at[b], gsem.at[b])
        return carry

    lax.fori_loop(0, NROUND, _round, 0)

    def _ddrain(i, carry):
        pltpu.make_async_copy(ones, dacc.at[didx.at[0]], dsem).wait()
        return carry

    lax.fori_loop(0, NCHUNK, _ddrain, 0)
    plsc.subcore_barrier()
    pltpu.sync_copy(acc.at[pl.ds(s * RPT, RPT)],
                    out_hbm.at[c, pl.ds(s * RPT, RPT)])
    pltpu.sync_copy(dacc.at[pl.ds(s * RPT, RPT)],
                    deg_hbm.at[c, pl.ds(s * RPT, RPT)])


def _ln(x, g, b, eps=1e-5):
    m = jnp.mean(x, axis=-1, keepdims=True)
    v = jnp.mean((x - m) ** 2, axis=-1, keepdims=True)
    return (x - m) / jnp.sqrt(v + eps) * g + b


_SQRT_HALF = 0.7071067811865476


def _gelu(x):
    return 0.5 * x * (1.0 + lax.erf(x * _SQRT_HALF))


def _pre_body(x_ref, pg, pb, gg, gb, h_ref, z0_ref):
    x = x_ref[...]
    h = _gelu(_ln(x, pg[...], pb[...]))
    h_ref[...] = h
    z0_ref[...] = jax.nn.relu(_ln(h[:, DG:], gg[...], gb[...]))


BR = 2000
_GRID = N // BR

def _bs_nd(d):
    return pl.BlockSpec((BR, d), lambda i: (i, 0))

def _bs_w(shape):
    return pl.BlockSpec(shape, lambda i: tuple(0 for _ in shape))

_bs_aggp = pl.BlockSpec((NC, BR, DG), lambda i: (0, i, 0))
_bs_degp = pl.BlockSpec((NC, BR, DW), lambda i: (0, i, 0))


_tc_pre = pl.pallas_call(
    _pre_body,
    grid=(_GRID,),
    in_specs=[_bs_nd(D), _bs_w((1, D)), _bs_w((1, D)),
              _bs_w((1, DG)), _bs_w((1, DG))],
    out_specs=(_bs_nd(D), _bs_nd(DG)),
    out_shape=(jax.ShapeDtypeStruct((N, D), jnp.float32),
               jax.ShapeDtypeStruct((N, DG), jnp.float32)),
)


def _agg_from_partials(aggp_ref, degp_ref):
    agg = aggp_ref[0] + aggp_ref[1]
    deg = degp_ref[0] + degp_ref[1]
    deg = jnp.maximum(deg, 1.0)
    return agg / deg[:, :1]


def _mid_body(h_ref, z_ref, aggp_ref, degp_ref, wl, bl, wr, br, gg, gb,
              y0_ref, z1_ref):
    z = z_ref[...]
    agg = _agg_from_partials(aggp_ref, degp_ref)
    conv = (jnp.dot(agg, wl[...], preferred_element_type=jnp.float32) + bl[...]
            + jnp.dot(z, wr[...], preferred_element_type=jnp.float32) + br[...])
    y0 = h_ref[:, :DG] + conv
    y0_ref[...] = y0
    z1_ref[...] = jax.nn.relu(_ln(y0, gg[...], gb[...]))


_tc_mid = pl.pallas_call(
    _mid_body,
    grid=(_GRID,),
    in_specs=[_bs_nd(D), _bs_nd(DG), _bs_aggp, _bs_degp,
              _bs_w((DG, DG)), _bs_w((1, DG)), _bs_w((DG, DG)), _bs_w((1, DG)),
              _bs_w((1, DG)), _bs_w((1, DG))],
    out_specs=(_bs_nd(DG), _bs_nd(DG)),
    out_shape=(jax.ShapeDtypeStruct((N, DG), jnp.float32),
               jax.ShapeDtypeStruct((N, DG), jnp.float32)),
)


def _post_x(h_ref, y0_ref, z1_ref, aggp_ref, degp_ref, wl, bl, wr, br,
            xres_ref, og, ob):
    z1 = z1_ref[...]
    agg = _agg_from_partials(aggp_ref, degp_ref)
    conv = (jnp.dot(agg, wl[...], preferred_element_type=jnp.float32) + bl[...]
            + jnp.dot(z1, wr[...], preferred_element_type=jnp.float32) + br[...])
    y1 = h_ref[:, DG:] + conv
    hcat = jnp.concatenate([y0_ref[...], y1], axis=-1)
    return _ln(hcat + xres_ref[...], og[...], ob[...])


def _postpre_body(h_ref, y0_ref, z1_ref, aggp_ref, degp_ref, wl, bl, wr, br,
                  xres_ref, og, ob, pg, pb, gg, gb,
                  xn_ref, hn_ref, zn_ref):
    xn = _post_x(h_ref, y0_ref, z1_ref, aggp_ref, degp_ref, wl, bl, wr, br,
                 xres_ref, og, ob)
    xn_ref[...] = xn
    hn = _gelu(_ln(xn, pg[...], pb[...]))
    hn_ref[...] = hn
    zn_ref[...] = jax.nn.relu(_ln(hn[:, DG:], gg[...], gb[...]))


_tc_postpre = pl.pallas_call(
    _postpre_body,
    grid=(_GRID,),
    in_specs=[_bs_nd(D), _bs_nd(DG), _bs_nd(DG), _bs_aggp, _bs_degp,
              _bs_w((DG, DG)), _bs_w((1, DG)), _bs_w((DG, DG)), _bs_w((1, DG)),
              _bs_nd(D), _bs_w((1, D)), _bs_w((1, D)),
              _bs_w((1, D)), _bs_w((1, D)), _bs_w((1, DG)), _bs_w((1, DG))],
    out_specs=(_bs_nd(D), _bs_nd(D), _bs_nd(DG)),
    out_shape=(jax.ShapeDtypeStruct((N, D), jnp.float32),
               jax.ShapeDtypeStruct((N, D), jnp.float32),
               jax.ShapeDtypeStruct((N, DG), jnp.float32)),
)


def _postgelu_body(h_ref, y0_ref, z1_ref, aggp_ref, degp_ref, wl, bl, wr, br,
                   xres_ref, og, ob, out_ref):
    xn = _post_x(h_ref, y0_ref, z1_ref, aggp_ref, degp_ref, wl, bl, wr, br,
                 xres_ref, og, ob)
    out_ref[...] = _gelu(xn)


_tc_postgelu = pl.pallas_call(
    _postgelu_body,
    grid=(_GRID,),
    in_specs=[_bs_nd(D), _bs_nd(DG), _bs_nd(DG), _bs_aggp, _bs_degp,
              _bs_w((DG, DG)), _bs_w((1, DG)), _bs_w((DG, DG)), _bs_w((1, DG)),
              _bs_nd(D), _bs_w((1, D)), _bs_w((1, D))],
    out_specs=_bs_nd(D),
    out_shape=jax.ShapeDtypeStruct((N, D), jnp.float32),
)


def kernel(x, edge_index, edge_weights, edge_attr, pre_ln_g, pre_ln_b,
           grp_ln_g, grp_ln_b, Wl, bl, Wr, br, post_ln_g, post_ln_b,
           et_ln_g, et_ln_b, et_W, et_b, en_g, en_b):
    src = edge_index[0].reshape(E // K, K)
    dst = edge_index[1].reshape(E // K, K)
    Wlt = jnp.swapaxes(Wl, -1, -2)
    Wrt = jnp.swapaxes(Wr, -1, -2)
    xres = x
    h, z0 = _tc_pre(x, pre_ln_g[0][None], pre_ln_b[0][None],
                    grp_ln_g[0, 0][None], grp_ln_b[0, 0][None])
    aggp, degp = _sc_segsum_deg(src, dst, z0)
    for l in range(L):
        y0, z1 = _tc_mid(h, z0, aggp, degp, Wlt[l, 0], bl[l, 0][None],
                         Wrt[l, 0], br[l, 0][None],
                         grp_ln_g[l, 1][None], grp_ln_b[l, 1][None])
        aggp = _sc_segsum(src, dst, z1)
        if l < L - 1:
            xres, h, z0 = _tc_postpre(
                h, y0, z1, aggp, degp, Wlt[l, 1], bl[l, 1][None],
                Wrt[l, 1], br[l, 1][None], xres,
                post_ln_g[l][None], post_ln_b[l][None],
                pre_ln_g[l + 1][None], pre_ln_b[l + 1][None],
                grp_ln_g[l + 1, 0][None], grp_ln_b[l + 1, 0][None])
            aggp = _sc_segsum(src, dst, z0)
        else:
            x = _tc_postgelu(h, y0, z1, aggp, degp, Wlt[l, 1], bl[l, 1][None],
                             Wrt[l, 1], br[l, 1][None], xres,
                             post_ln_g[l][None], post_ln_b[l][None])
    return x


# speedup vs baseline: 14.2532x; 1.0155x over previous
import functools

import jax
import jax.numpy as jnp
from jax import lax
from jax.experimental import pallas as pl
from jax.experimental.pallas import tpu as pltpu
from jax.experimental.pallas import tpu_sc as plsc

N, E, D, DG, L = 10000, 320000, 128, 64, 4
NPAD = 10240
NC, NS = 2, 16
NW = NC * NS
EPW = E // NW
K = 125
NCHUNK = EPW // K
NBUF = 5
NROUND = NCHUNK // NBUF
RPT = NPAD // NS
DW = 16

_mesh = plsc.VectorSubcoreMesh(
    core_axis_name="c", subcore_axis_name="s", num_cores=NC, num_subcores=NS)
_sc_params = pltpu.CompilerParams(use_tc_tiling_on_sc=False)


@functools.partial(
    pl.kernel,
    out_type=jax.ShapeDtypeStruct((NC, NPAD, DG), jnp.float32),
    mesh=_mesh,
    compiler_params=_sc_params,
    scratch_types=[
        pltpu.VMEM((NCHUNK, K), jnp.int32),
        pltpu.VMEM((NCHUNK, K), jnp.int32),
        pltpu.VMEM((NBUF, K, DG), jnp.float32),
        pltpu.VMEM((64, DG), jnp.float32),
        pltpu.VMEM_SHARED((NPAD, DG), jnp.float32),
        pltpu.SemaphoreType.DMA((NBUF,)),
        pltpu.SemaphoreType.DMA((NBUF,)),
        pltpu.SemaphoreType.DMA,
        pltpu.SemaphoreType.DMA,
    ],
)
def _sc_segsum(edge_hbm, z_hbm, out_hbm, sidx, didx, rows, zbuf, acc,
               gsem, ssem, isem, zsem):
    c = lax.axis_index("c")
    s = lax.axis_index("s")
    wid = s * NC + c
    zero16 = jnp.zeros((16,), jnp.float32)

    ild0 = pltpu.async_copy(edge_hbm.at[0, pl.ds(wid * NCHUNK, NCHUNK)], sidx,
                            isem)
    ild1 = pltpu.async_copy(edge_hbm.at[1, pl.ds(wid * NCHUNK, NCHUNK)], didx,
                            isem)

    def _zb(i, carry):
        zbuf[i // (DG // 16), pl.ds((i % (DG // 16)) * 16, 16)] = zero16
        return carry

    lax.fori_loop(0, 64 * (DG // 16), _zb, 0)

    def _zc(j, carry):
        pltpu.async_copy(zbuf, acc.at[pl.ds(s * RPT + j * 64, 64)], zsem)
        return carry

    lax.fori_loop(0, RPT // 64, _zc, 0)

    def _zd(j, carry):
        pltpu.make_async_copy(zbuf, acc.at[pl.ds(0, 64)], zsem).wait()
        return carry

    lax.fori_loop(0, RPT // 64, _zd, 0)
    ild0.wait()
    ild1.wait()
    plsc.subcore_barrier()

    for b in range(NBUF):
        pltpu.async_copy(z_hbm.at[sidx.at[b]], rows.at[b], gsem.at[b])

    def _round(g, carry):
        descs = []
        for b in range(NBUF):
            pltpu.make_async_copy(z_hbm.at[pl.ds(0, K)], rows.at[b],
                                  gsem.at[b]).wait()
            descs.append(pltpu.async_copy(rows.at[b], acc.at[didx.at[g * NBUF + b]],
                                          ssem.at[b], add=True))
        for b in range(NBUF):
            descs[b].wait()

            @pl.when(g + 1 < NROUND)
            def _():
                pltpu.async_copy(z_hbm.at[sidx.at[(g + 1) * NBUF + b]],
                                 rows.at[b], gsem.at[b])
        return carry

    lax.fori_loop(0, NROUND, _round, 0)
    plsc.subcore_barrier()
    pltpu.sync_copy(acc.at[pl.ds(s * RPT, RPT)],
                    out_hbm.at[c, pl.ds(s * RPT, RPT)])


@functools.partial(
    pl.kernel,
    out_type=(jax.ShapeDtypeStruct((NC, NPAD, DG), jnp.float32),
              jax.ShapeDtypeStruct((NC, NPAD, DW), jnp.float32)),
    mesh=_mesh,
    compiler_params=_sc_params,
    scratch_types=[
        pltpu.VMEM((NCHUNK, K), jnp.int32),
        pltpu.VMEM((NCHUNK, K), jnp.int32),
        pltpu.VMEM((NBUF, K, DG), jnp.float32),
        pltpu.VMEM((64, DG), jnp.float32),
        pltpu.VMEM((K, DW), jnp.float32),
        pltpu.VMEM((64, DW), jnp.float32),
        pltpu.VMEM_SHARED((NPAD, DG), jnp.float32),
        pltpu.VMEM_SHARED((NPAD, DW), jnp.float32),
        pltpu.SemaphoreType.DMA((NBUF,)),
        pltpu.SemaphoreType.DMA((NBUF,)),
        pltpu.SemaphoreType.DMA,
        pltpu.SemaphoreType.DMA,
        pltpu.SemaphoreType.DMA,
    ],
)
def _sc_segsum_deg(edge_hbm, z_hbm, out_hbm, deg_hbm, sidx, didx, rows,
                   zbuf, ones, zbuf2, acc, dacc, gsem, ssem, isem, zsem, dsem):
    c = lax.axis_index("c")
    s = lax.axis_index("s")
    wid = s * NC + c
    zero16 = jnp.zeros((16,), jnp.float32)
    one16 = jnp.ones((16,), jnp.float32)

    ild0 = pltpu.async_copy(edge_hbm.at[0, pl.ds(wid * NCHUNK, NCHUNK)], sidx,
                            isem)
    ild1 = pltpu.async_copy(edge_hbm.at[1, pl.ds(wid * NCHUNK, NCHUNK)], didx,
                            isem)

    def _zb(i, carry):
        zbuf[i // (DG // 16), pl.ds((i % (DG // 16)) * 16, 16)] = zero16
        return carry

    lax.fori_loop(0, 64 * (DG // 16), _zb, 0)

    def _fill(i, carry):
        ones[i, :] = one16
        zbuf2[i % 64, :] = zero16
        return carry

    lax.fori_loop(0, K, _fill, 0)

    def _zc(j, carry):
        pltpu.async_copy(zbuf, acc.at[pl.ds(s * RPT + j * 64, 64)], zsem)
        pltpu.async_copy(zbuf2, dacc.at[pl.ds(s * RPT + j * 64, 64)], zsem)
        return carry

    lax.fori_loop(0, RPT // 64, _zc, 0)

    def _zd(j, carry):
        pltpu.make_async_copy(zbuf, acc.at[pl.ds(0, 64)], zsem).wait()
        pltpu.make_async_copy(zbuf2, dacc.at[pl.ds(0, 64)], zsem).wait()
        return carry

    lax.fori_loop(0, RPT // 64, _zd, 0)
    ild0.wait()
    ild1.wait()
    plsc.subcore_barrier()

    for b in range(NBUF):
        pltpu.async_copy(z_hbm.at[sidx.at[b]], rows.at[b], gsem.at[b])

    def _round(g, carry):
        descs = []
        for b in range(NBUF):
            i = g * NBUF + b
            pltpu.make_async_copy(z_hbm.at[pl.ds(0, K)], rows.at[b],
                                  gsem.at[b]).wait()
            descs.append(pltpu.async_copy(rows.at[b], acc.at[didx.at[i]],
                                          ssem.at[b], add=True))
            pltpu.async_copy(ones, dacc.at[didx.at[i]], dsem, add=True)
        for b in range(NBUF):
            descs[b].wait()

            @pl.when(g + 1 < NROUND)
            def _():
                pltpu.async_copy(z_hbm.at[sidx.at[(g + 1) * NBUF + b]],
                                 rows.at[b], gsem.at[b])
        return carry

    lax.fori_loop(0, NROUND, _round, 0)

    def _ddrain(i, carry):
        pltpu.make_async_copy(ones, dacc.at[didx.at[0]], dsem).wait()
        return carry

    lax.fori_loop(0, NCHUNK, _ddrain, 0)
    plsc.subcore_barrier()
    pltpu.sync_copy(acc.at[pl.ds(s * RPT, RPT)],
                    out_hbm.at[c, pl.ds(s * RPT, RPT)])
    pltpu.sync_copy(dacc.at[pl.ds(s * RPT, RPT)],
                    deg_hbm.at[c, pl.ds(s * RPT, RPT)])


def _ln(x, g, b, eps=1e-5):
    m = jnp.mean(x, axis=-1, keepdims=True)
    v = jnp.mean((x - m) ** 2, axis=-1, keepdims=True)
    return (x - m) / jnp.sqrt(v + eps) * g + b


_SQRT_HALF = 0.7071067811865476


def _gelu(x):
    return 0.5 * x * (1.0 + lax.erf(x * _SQRT_HALF))


def _pre_body(x_ref, pg, pb, gg, gb, h_ref, z0_ref):
    x = x_ref[...]
    h = _gelu(_ln(x, pg[...], pb[...]))
    h_ref[...] = h
    z0_ref[...] = jax.nn.relu(_ln(h[:, DG:], gg[...], gb[...]))


BR = 2000
_GRID = N // BR

def _bs_nd(d):
    return pl.BlockSpec((BR, d), lambda i: (i, 0))

def _bs_w(shape):
    return pl.BlockSpec(shape, lambda i: tuple(0 for _ in shape))

_bs_aggp = pl.BlockSpec((NC, BR, DG), lambda i: (0, i, 0))
_bs_degp = pl.BlockSpec((NC, BR, DW), lambda i: (0, i, 0))


_tc_pre = pl.pallas_call(
    _pre_body,
    grid=(_GRID,),
    in_specs=[_bs_nd(D), _bs_w((1, D)), _bs_w((1, D)),
              _bs_w((1, DG)), _bs_w((1, DG))],
    out_specs=(_bs_nd(D), _bs_nd(DG)),
    out_shape=(jax.ShapeDtypeStruct((N, D), jnp.float32),
               jax.ShapeDtypeStruct((N, DG), jnp.float32)),
)


def _agg_from_partials(aggp_ref, degp_ref):
    agg = aggp_ref[0] + aggp_ref[1]
    deg = degp_ref[0] + degp_ref[1]
    deg = jnp.maximum(deg, 1.0)
    return agg / deg[:, :1]


def _mid_body(h_ref, z_ref, aggp_ref, degp_ref, wl, bl, wr, br, gg, gb,
              y0_ref, z1_ref):
    z = z_ref[...]
    agg = _agg_from_partials(aggp_ref, degp_ref)
    conv = (jnp.dot(agg, wl[...], preferred_element_type=jnp.float32) + bl[...]
            + jnp.dot(z, wr[...], preferred_element_type=jnp.float32) + br[...])
    y0 = h_ref[:, :DG] + conv
    y0_ref[...] = y0
    z1_ref[...] = jax.nn.relu(_ln(y0, gg[...], gb[...]))


_tc_mid = pl.pallas_call(
    _mid_body,
    grid=(_GRID,),
    in_specs=[_bs_nd(D), _bs_nd(DG), _bs_aggp, _bs_degp,
              _bs_w((DG, DG)), _bs_w((1, DG)), _bs_w((DG, DG)), _bs_w((1, DG)),
              _bs_w((1, DG)), _bs_w((1, DG))],
    out_specs=(_bs_nd(DG), _bs_nd(DG)),
    out_shape=(jax.ShapeDtypeStruct((N, DG), jnp.float32),
               jax.ShapeDtypeStruct((N, DG), jnp.float32)),
)


def _post_x(h_ref, y0_ref, z1_ref, aggp_ref, degp_ref, wl, bl, wr, br,
            xres_ref, og, ob):
    z1 = z1_ref[...]
    agg = _agg_from_partials(aggp_ref, degp_ref)
    conv = (jnp.dot(agg, wl[...], preferred_element_type=jnp.float32) + bl[...]
            + jnp.dot(z1, wr[...], preferred_element_type=jnp.float32) + br[...])
    y1 = h_ref[:, DG:] + conv
    hcat = jnp.concatenate([y0_ref[...], y1], axis=-1)
    return _ln(hcat + xres_ref[...], og[...], ob[...])


def _postpre_body(h_ref, y0_ref, z1_ref, aggp_ref, degp_ref, wl, bl, wr, br,
                  xres_ref, og, ob, pg, pb, gg, gb,
                  xn_ref, hn_ref, zn_ref):
    xn = _post_x(h_ref, y0_ref, z1_ref, aggp_ref, degp_ref, wl, bl, wr, br,
                 xres_ref, og, ob)
    xn_ref[...] = xn
    hn = _gelu(_ln(xn, pg[...], pb[...]))
    hn_ref[...] = hn
    zn_ref[...] = jax.nn.relu(_ln(hn[:, DG:], gg[...], gb[...]))


_tc_postpre = pl.pallas_call(
    _postpre_body,
    grid=(_GRID,),
    in_specs=[_bs_nd(D), _bs_nd(DG), _bs_nd(DG), _bs_aggp, _bs_degp,
              _bs_w((DG, DG)), _bs_w((1, DG)), _bs_w((DG, DG)), _bs_w((1, DG)),
              _bs_nd(D), _bs_w((1, D)), _bs_w((1, D)),
              _bs_w((1, D)), _bs_w((1, D)), _bs_w((1, DG)), _bs_w((1, DG))],
    out_specs=(_bs_nd(D), _bs_nd(D), _bs_nd(DG)),
    out_shape=(jax.ShapeDtypeStruct((N, D), jnp.float32),
               jax.ShapeDtypeStruct((N, D), jnp.float32),
               jax.ShapeDtypeStruct((N, DG), jnp.float32)),
)


def _postgelu_body(h_ref, y0_ref, z1_ref, aggp_ref, degp_ref, wl, bl, wr, br,
                   xres_ref, og, ob, out_ref):
    xn = _post_x(h_ref, y0_ref, z1_ref, aggp_ref, degp_ref, wl, bl, wr, br,
                 xres_ref, og, ob)
    out_ref[...] = _gelu(xn)


_tc_postgelu = pl.pallas_call(
    _postgelu_body,
    grid=(_GRID,),
    in_specs=[_bs_nd(D), _bs_nd(DG), _bs_nd(DG), _bs_aggp, _bs_degp,
              _bs_w((DG, DG)), _bs_w((1, DG)), _bs_w((DG, DG)), _bs_w((1, DG)),
              _bs_nd(D), _bs_w((1, D)), _bs_w((1, D))],
    out_specs=_bs_nd(D),
    out_shape=jax.ShapeDtypeStruct((N, D), jnp.float32),
)


def kernel(x, edge_index, edge_weights, edge_attr, pre_ln_g, pre_ln_b,
           grp_ln_g, grp_ln_b, Wl, bl, Wr, br, post_ln_g, post_ln_b,
           et_ln_g, et_ln_b, et_W, et_b, en_g, en_b):
    edge2d = edge_index.reshape(2, E // K, K)
    Wlt = jnp.swapaxes(Wl, -1, -2)
    Wrt = jnp.swapaxes(Wr, -1, -2)
    xres = x
    h, z0 = _tc_pre(x, pre_ln_g[0][None], pre_ln_b[0][None],
                    grp_ln_g[0, 0][None], grp_ln_b[0, 0][None])
    aggp, degp = _sc_segsum_deg(edge2d, z0)
    for l in range(L):
        y0, z1 = _tc_mid(h, z0, aggp, degp, Wlt[l, 0], bl[l, 0][None],
                         Wrt[l, 0], br[l, 0][None],
                         grp_ln_g[l, 1][None], grp_ln_b[l, 1][None])
        aggp = _sc_segsum(edge2d, z1)
        if l < L - 1:
            xres, h, z0 = _tc_postpre(
                h, y0, z1, aggp, degp, Wlt[l, 1], bl[l, 1][None],
                Wrt[l, 1], br[l, 1][None], xres,
                post_ln_g[l][None], post_ln_b[l][None],
                pre_ln_g[l + 1][None], pre_ln_b[l + 1][None],
                grp_ln_g[l + 1, 0][None], grp_ln_b[l + 1, 0][None])
            aggp = _sc_segsum(edge2d, z0)
        else:
            x = _tc_postgelu(h, y0, z1, aggp, degp, Wlt[l, 1], bl[l, 1][None],
                             Wrt[l, 1], br[l, 1][None], xres,
                             post_ln_g[l][None], post_ln_b[l][None])
    return x
